# Initial kernel scaffold; baseline (speedup 1.0000x reference)
import jax
import jax.numpy as jnp
from jax.experimental import pallas as pl


def kernel(img):
    raise NotImplementedError("write your pallas kernel here")


# speedup vs baseline: 8.7110x; 8.7110x over previous
import functools

import numpy as np
import jax
import jax.numpy as jnp
from jax.experimental import pallas as pl

_H, _W, _C = 512, 512, 3
_SIGMA = 0.4
_LANES = _W * _C


def _blur_consts():
    radius = int(4.0 * _SIGMA + 0.5)
    x = np.arange(-radius, radius + 1)
    k = np.exp(-0.5 * (x / _SIGMA) ** 2)
    k = k / k.sum()
    a2, a1, a0 = float(k[0]), float(k[1]), float(k[2])
    m = np.array(
        [
            [a0 + a1 + a2, a1, a2],
            [a1 + a2, a0, a1 + a2],
            [a2, a1, a0 + a1 + a2],
        ]
    )
    return a1, a0, m @ m


@functools.lru_cache(maxsize=1)
def _swap_mask():
    rng = np.random.default_rng(0)
    d = rng.integers(-1, 1, size=(_H - 2, _W - 2, 2))
    dx = np.zeros((_H, _W), np.int8)
    dy = np.zeros((_H, _W), np.int8)
    dx[2:, 2:] = d[::-1, ::-1, 0]
    dy[2:, 2:] = d[::-1, ::-1, 1]
    m = (dx == -1).astype(np.int8) | ((dy == -1).astype(np.int8) << 1)
    m = np.repeat(m, _C, axis=1)
    return jnp.asarray(m)


def _row_up(x):
    return jnp.concatenate([x[:1], x[:-1]], axis=0)


def _row_dn(x):
    return jnp.concatenate([x[1:], x[-1:]], axis=0)


def _pix_left(x):
    return jnp.concatenate([x[:, :_C], x[:, :-_C]], axis=1)


def _pix_right(x):
    return jnp.concatenate([x[:, _C:], x[:, -_C:]], axis=1)


def _lane_shift(x, o):
    if o == 0:
        return x
    return jnp.concatenate([x[:, o:], x[:, :o]], axis=1)


def _body(a1, a0, m2, img_ref, mask_ref, out_ref):
    x = img_ref[...]
    m = mask_ref[...]

    x = a0 * x + a1 * (_row_up(x) + _row_dn(x))
    x = a0 * x + a1 * (_pix_left(x) + _pix_right(x))

    bdx = (m & 1) != 0
    bdy = (m & 2) != 0
    xu = _row_up(x)
    t0 = jnp.where(bdx, _pix_left(x), x)
    t1 = jnp.where(bdx, _pix_left(xu), xu)
    x = jnp.where(bdy, t1, t0)

    x = a0 * x + a1 * (_row_up(x) + _row_dn(x))
    x = a0 * x + a1 * (_pix_left(x) + _pix_right(x))

    c = jax.lax.broadcasted_iota(jnp.int32, (1, _LANES), 1) % _C
    y = jnp.zeros_like(x)
    for o in range(-2, 3):
        w_c = [m2[ci, ci + o] if 0 <= ci + o < _C else 0.0 for ci in range(_C)]
        if all(v == 0.0 for v in w_c):
            continue
        wv = jnp.where(c == 0, w_c[0], jnp.where(c == 1, w_c[1], w_c[2]))
        y = y + wv.astype(x.dtype) * _lane_shift(x, o)

    out_ref[...] = jnp.clip(y, 0.0, 1.0)


@jax.jit
def kernel(img):
    a1, a0, m2 = _blur_consts()
    body = functools.partial(_body, a1, a0, m2)
    out = pl.pallas_call(
        body,
        out_shape=jax.ShapeDtypeStruct((_H, _LANES), jnp.float32),
    )(img.reshape(_H, _LANES), _swap_mask())
    return out.reshape(_H, _W, _C)


# speedup vs baseline: 9.5323x; 1.0943x over previous
import functools

import numpy as np
import jax
import jax.numpy as jnp
from jax.experimental import pallas as pl

_H, _W, _C = 512, 512, 3
_SIGMA = 0.4
_LANES = _W * _C
_B = 128
_NB = _H // _B
_S = 8
_SPB = _B // _S


@functools.lru_cache(maxsize=1)
def _consts():
    radius = int(4.0 * _SIGMA + 0.5)
    xs = np.arange(-radius, radius + 1)
    k = np.exp(-0.5 * (xs / _SIGMA) ** 2)
    k = k / k.sum()
    a2, a1, a0 = float(k[0]), float(k[1]), float(k[2])
    m = np.array(
        [
            [a0 + a1 + a2, a1, a2],
            [a1 + a2, a0, a1 + a2],
            [a2, a1, a0 + a1 + a2],
        ]
    )
    m2 = m @ m
    m2[0, 1] += m2[0, 2]
    m2[2, 1] += m2[2, 0]
    wrows = np.zeros((8, _LANES), np.float32)
    for r, o in enumerate((-1, 0, 1)):
        for l in range(_LANES):
            c = l % _C
            if 0 <= c + o < _C:
                wrows[r, l] = m2[c, c + o]
    return a1, a0, jnp.asarray(wrows)


@functools.lru_cache(maxsize=1)
def _swap_mask():
    rng = np.random.default_rng(0)
    d = rng.integers(-1, 1, size=(_H - 2, _W - 2, 2))
    dx = np.zeros((_H, _W), np.int8)
    dy = np.zeros((_H, _W), np.int8)
    dx[2:, 2:] = d[::-1, ::-1, 0]
    dy[2:, 2:] = d[::-1, ::-1, 1]
    m = (dx == -1).astype(np.int8) | ((dy == -1).astype(np.int8) << 1)
    m = np.repeat(m, _C, axis=1)
    win = np.zeros((_NB, _S + _B + _S, _LANES), np.int8)
    for i in range(_NB):
        rows = np.clip(np.arange(_B * i - _S, _B * i + _B + _S), 0, _H - 1)
        win[i] = m[rows]
    t = (_S + _B + _S) // 3
    packed = win[:, :t] | (win[:, t:2 * t] << 2) | (win[:, 2 * t:] << 4)
    return jnp.asarray(packed)


def _row_up(x):
    return jnp.concatenate([x[:1], x[:-1]], axis=0)


def _row_dn(x):
    return jnp.concatenate([x[1:], x[-1:]], axis=0)


def _pix_left(x):
    return jnp.concatenate([x[:, :_C], x[:, :-_C]], axis=1)


def _pix_right(x):
    return jnp.concatenate([x[:, _C:], x[:, -_C:]], axis=1)


def _lane_shift(x, o):
    return jnp.concatenate([x[:, o:], x[:, :o]], axis=1)


def _body(a1, a0, cur_ref, ha_ref, hb_ref, m_ref, w_ref, out_ref):
    i = pl.program_id(0)

    x = jnp.concatenate([ha_ref[...], cur_ref[...], hb_ref[...]], axis=0)
    p = m_ref[...].reshape((_S + _B + _S) // 3, _LANES).astype(jnp.int32)
    m = jnp.concatenate([p & 3, (p >> 2) & 3, (p >> 4) & 3], axis=0)

    r = jax.lax.broadcasted_iota(jnp.int32, (_S + _B + _S, 1), 0)
    top = (i == 0) & (r < _S)
    bot = (i == _NB - 1) & (r >= _S + _B)

    def _edge_fix(v):
        v = jnp.where(top, v[_S:_S + 1], v)
        return jnp.where(bot, v[_S + _B - 1:_S + _B], v)

    x = _edge_fix(x)

    x = a0 * x + a1 * (_row_up(x) + _row_dn(x))
    x = a0 * x + a1 * (_pix_left(x) + _pix_right(x))
    x = _edge_fix(x)

    bdx = (m & 1) != 0
    bdy = (m & 2) != 0
    xl = _pix_left(x)
    t0 = jnp.where(bdx, xl, x)
    t1 = jnp.where(bdx, _row_up(xl), _row_up(x))
    x = jnp.where(bdy, t1, t0)
    x = _edge_fix(x)

    x = a0 * x + a1 * (_row_up(x) + _row_dn(x))
    x = a0 * x + a1 * (_pix_left(x) + _pix_right(x))

    y = w_ref[1:2] * x
    y = y + w_ref[0:1] * _lane_shift(x, -1)
    y = y + w_ref[2:3] * _lane_shift(x, 1)

    out_ref[...] = jnp.minimum(jnp.maximum(y[_S:_S + _B], 0.0), 1.0)


@jax.jit
def kernel(img):
    a1, a0, wrows = _consts()
    body = functools.partial(_body, a1, a0)
    blk = lambda i: (i, 0)
    above = lambda i: (jnp.maximum(_SPB * i - 1, 0), 0)
    below = lambda i: (jnp.minimum(_SPB * i + _SPB, _H // _S - 1), 0)
    out = pl.pallas_call(
        body,
        grid=(_NB,),
        in_specs=[
            pl.BlockSpec((_B, _LANES), blk),
            pl.BlockSpec((_S, _LANES), above),
            pl.BlockSpec((_S, _LANES), below),
            pl.BlockSpec((1, (_S + _B + _S) // 3, _LANES),
                         lambda i: (i, 0, 0)),
            pl.BlockSpec((8, _LANES), lambda i: (0, 0)),
        ],
        out_specs=pl.BlockSpec((_B, _LANES), blk),
        out_shape=jax.ShapeDtypeStruct((_H, _LANES), jnp.float32),
    )(img.reshape(_H, _LANES), img.reshape(_H, _LANES),
      img.reshape(_H, _LANES), _swap_mask(), wrows)
    return out.reshape(_H, _W, _C)
